# Initial kernel scaffold; baseline (speedup 1.0000x reference)
import functools

import jax
import jax.numpy as jnp
from jax import lax
from jax.experimental import pallas as pl
from jax.experimental.pallas import tpu as pltpu
from jax.experimental.pallas import tpu_sc as plsc

N = 10000
E = 640000
NC = 2
NS = 16
NW = NC * NS
BLK = 128
NBLK = 157
EPAD = NW * NBLK * BLK
ROWS = 10016
SLICE = ROWS // NS

_MESH = plsc.VectorSubcoreMesh(core_axis_name="c", subcore_axis_name="s")


def _zero_rows(zbuf, nrows, f):
    def body(i, _):
        for f0 in range(0, f, 16):
            zbuf[i, f0:f0 + 16] = jnp.zeros((16,), jnp.float32)
        return 0
    lax.fori_loop(0, nrows, body, 0)


def _make_deg_kernel():
    @functools.partial(
        pl.kernel,
        out_type=jax.ShapeDtypeStruct((NC, ROWS, 16), jnp.float32),
        mesh=_MESH,
        scratch_types=[
            pltpu.VMEM((NBLK, BLK), jnp.int32),
            pltpu.VMEM((BLK, 16), jnp.float32),
            pltpu.VMEM((SLICE, 16), jnp.float32),
            pltpu.VMEM_SHARED((ROWS, 16), jnp.float32),
        ],
    )
    def deg_kernel(dstp_hbm, out_hbm, dst_v, ones_v, zbuf, acc):
        c = lax.axis_index("c")
        s = lax.axis_index("s")
        wid = c * NS + s
        _zero_rows(zbuf, SLICE, 16)
        pltpu.sync_copy(zbuf, acc.at[pl.ds(s * SLICE, SLICE)])
        e0 = jnp.where(lax.iota(jnp.int32, 16) == 0, 1.0, 0.0).astype(jnp.float32)
        def fill(i, _):
            ones_v[i, :] = e0
            return 0
        lax.fori_loop(0, BLK, fill, 0)
        pltpu.sync_copy(dstp_hbm.at[wid], dst_v)
        plsc.subcore_barrier()
        def body(j, _):
            pltpu.sync_copy(ones_v, acc.at[dst_v.at[j]], add=True)
            return 0
        lax.fori_loop(0, NBLK, body, 0)
        plsc.subcore_barrier()
        pltpu.sync_copy(acc.at[pl.ds(s * SLICE, SLICE)],
                        out_hbm.at[c, pl.ds(s * SLICE, SLICE)])

    return deg_kernel


def _make_scatter_kernel(f):
    @functools.partial(
        pl.kernel,
        out_type=jax.ShapeDtypeStruct((NC, ROWS, f), jnp.float32),
        mesh=_MESH,
        scratch_types=[
            pltpu.VMEM((NBLK, BLK), jnp.int32),
            pltpu.VMEM((NBLK, BLK), jnp.int32),
            pltpu.VMEM((BLK, f), jnp.float32),
            pltpu.VMEM((SLICE, f), jnp.float32),
            pltpu.VMEM_SHARED((ROWS, f), jnp.float32),
            pltpu.SemaphoreType.DMA,
        ],
    )
    def scatter_kernel(g_hbm, srcp_hbm, dstp_hbm, out_hbm,
                       src_v, dst_v, rows_v, zbuf, acc, sem):
        c = lax.axis_index("c")
        s = lax.axis_index("s")
        wid = c * NS + s
        _zero_rows(zbuf, SLICE, f)
        pltpu.sync_copy(zbuf, acc.at[pl.ds(s * SLICE, SLICE)])
        pltpu.sync_copy(srcp_hbm.at[wid], src_v)
        pltpu.sync_copy(dstp_hbm.at[wid], dst_v)
        plsc.subcore_barrier()
        def body(j, _):
            pltpu.async_copy(g_hbm.at[src_v.at[j]], rows_v, sem).wait()
            pltpu.sync_copy(rows_v, acc.at[dst_v.at[j]], add=True)
            return 0
        lax.fori_loop(0, NBLK, body, 0)
        plsc.subcore_barrier()
        pltpu.sync_copy(acc.at[pl.ds(s * SLICE, SLICE)],
                        out_hbm.at[c, pl.ds(s * SLICE, SLICE)])

    return scatter_kernel


_deg_kernel = _make_deg_kernel()
_scatter16 = _make_scatter_kernel(16)
_scatter32 = _make_scatter_kernel(32)


def _tc1_body(degp_ref, x_ref, w_ref, g_ref, dinv_ref):
    d = degp_ref[...]
    deg = d[0, :N, 0:1] + d[1, :N, 0:1] + 1.0
    dinv = lax.rsqrt(deg)
    dinv16 = jnp.broadcast_to(dinv, (N, 16))
    h = jnp.dot(x_ref[...], w_ref[...], preferred_element_type=jnp.float32)
    g_ref[...] = dinv16 * h
    dinv_ref[...] = dinv16


def _tc_mid_body(s_ref, g_ref, dinv_ref, w_ref, b_ref, o_ref):
    sp = s_ref[...]
    ssum = sp[0, :N, :] + sp[1, :N, :]
    dinv = dinv_ref[...]
    u = jnp.maximum(dinv * (ssum + g_ref[...]) + b_ref[...], 0.0)
    h = jnp.dot(u, w_ref[...], preferred_element_type=jnp.float32)
    fo = h.shape[1]
    dinv_o = jnp.broadcast_to(dinv[:, 0:1], (N, fo))
    o_ref[...] = dinv_o * h


def _tc_out_body(s_ref, g_ref, dinv_ref, b_ref, o_ref):
    sp = s_ref[...]
    ssum = sp[0, :N, :] + sp[1, :N, :]
    dinv = jnp.broadcast_to(dinv_ref[...][:, 0:1], (N, 32))
    o_ref[...] = jax.nn.sigmoid(dinv * (ssum + g_ref[...]) + b_ref[...])


def _tc1(degp, x, w1):
    return pl.pallas_call(
        _tc1_body,
        out_shape=[jax.ShapeDtypeStruct((N, 16), jnp.float32),
                   jax.ShapeDtypeStruct((N, 16), jnp.float32)],
    )(degp, x, w1)


def _tc_mid(s, g, dinv, w, b, fo):
    return pl.pallas_call(
        _tc_mid_body,
        out_shape=jax.ShapeDtypeStruct((N, fo), jnp.float32),
    )(s, g, dinv, w, b)


def _tc_out(s, g, dinv, b):
    return pl.pallas_call(
        _tc_out_body,
        out_shape=jax.ShapeDtypeStruct((N, 32), jnp.float32),
    )(s, g, dinv, b)


def kernel(x, edge_index, W1, b1, W2, b2, W3, b3):
    src = edge_index[0].astype(jnp.int32)
    dst = edge_index[1].astype(jnp.int32)
    npad = EPAD - E
    srcp = jnp.concatenate([src, jnp.zeros((npad,), jnp.int32)]).reshape(NW, NBLK, BLK)
    dstp = jnp.concatenate([dst, jnp.full((npad,), ROWS - 1, jnp.int32)]).reshape(NW, NBLK, BLK)

    degp = _deg_kernel(dstp)
    g1, dinv = _tc1(degp, x, W1)
    s1 = _scatter16(g1, srcp, dstp)
    g2 = _tc_mid(s1, g1, dinv, W2, b1.reshape(1, 16), 16)
    s2 = _scatter16(g2, srcp, dstp)
    g3 = _tc_mid(s2, g2, dinv, W3, b2.reshape(1, 16), 32)
    s3 = _scatter32(g3, srcp, dstp)
    return _tc_out(s3, g3, dinv, b3.reshape(1, 32))


# speedup vs baseline: 35.7805x; 35.7805x over previous
import functools

import jax
import jax.numpy as jnp
from jax import lax
from jax.experimental import pallas as pl
from jax.experimental.pallas import tpu as pltpu
from jax.experimental.pallas import tpu_sc as plsc

N = 10000
E = 640000
NC = 2
NS = 16
NW = NC * NS
BLK = 128
NBLK = 157
EPAD = NW * NBLK * BLK
ROWS = 10112
SLICE = ROWS // NS

_MESH = plsc.VectorSubcoreMesh(core_axis_name="c", subcore_axis_name="s")


def _zero_rows(zbuf, nrows, f):
    def body(i, _):
        for f0 in range(0, f, 16):
            zbuf[i, f0:f0 + 16] = jnp.zeros((16,), jnp.float32)
        return 0
    lax.fori_loop(0, nrows, body, 0)


HR = ROWS // 128


def _make_deg_kernel():
    @functools.partial(
        pl.kernel,
        out_type=jax.ShapeDtypeStruct((NW, ROWS), jnp.float32),
        mesh=_MESH,
        scratch_types=[
            pltpu.VMEM((NBLK, BLK), jnp.int32),
            pltpu.VMEM((ROWS,), jnp.float32),
        ],
        compiler_params=pltpu.CompilerParams(needs_layout_passes=False),
    )
    def deg_kernel(dstp_hbm, out_hbm, dst_v, hist):
        c = lax.axis_index("c")
        s = lax.axis_index("s")
        wid = c * NS + s
        def z(i, _):
            hist[pl.ds(i * 16, 16)] = jnp.zeros((16,), jnp.float32)
            return 0
        lax.fori_loop(0, ROWS // 16, z, 0)
        pltpu.sync_copy(dstp_hbm.at[wid], dst_v)
        ones = jnp.ones((16,), jnp.float32)
        def body(j, _):
            for k in range(0, BLK, 16):
                d = dst_v[j, k:k + 16]
                plsc.addupdate_scatter(hist, [d], ones)
            return 0
        lax.fori_loop(0, NBLK, body, 0)
        pltpu.sync_copy(hist, out_hbm.at[wid])

    return deg_kernel


def _make_scatter_kernel(f):
    @functools.partial(
        pl.kernel,
        out_type=jax.ShapeDtypeStruct((NC, ROWS, f), jnp.float32),
        mesh=_MESH,
        scratch_types=[
            pltpu.VMEM((NBLK, BLK), jnp.int32),
            pltpu.VMEM((NBLK, BLK), jnp.int32),
            pltpu.VMEM((BLK, f), jnp.float32),
            pltpu.VMEM((SLICE, f), jnp.float32),
            pltpu.VMEM_SHARED((ROWS, f), jnp.float32),
            pltpu.SemaphoreType.DMA,
        ],
        compiler_params=pltpu.CompilerParams(use_tc_tiling_on_sc=False),
    )
    def scatter_kernel(g_hbm, srcp_hbm, dstp_hbm, out_hbm,
                       src_v, dst_v, rows_v, zbuf, acc, sem):
        c = lax.axis_index("c")
        s = lax.axis_index("s")
        wid = c * NS + s
        _zero_rows(zbuf, SLICE, f)
        pltpu.sync_copy(zbuf, acc.at[pl.ds(s * SLICE, SLICE)])
        pltpu.sync_copy(srcp_hbm.at[wid], src_v)
        pltpu.sync_copy(dstp_hbm.at[wid], dst_v)
        plsc.subcore_barrier()
        def body(j, _):
            pltpu.async_copy(g_hbm.at[src_v.at[j]], rows_v, sem).wait()
            pltpu.sync_copy(rows_v, acc.at[dst_v.at[j]], add=True)
            return 0
        lax.fori_loop(0, NBLK, body, 0)
        plsc.subcore_barrier()
        pltpu.sync_copy(acc.at[pl.ds(s * SLICE, SLICE)],
                        out_hbm.at[c, pl.ds(s * SLICE, SLICE)])

    return scatter_kernel


_deg_kernel = _make_deg_kernel()
_scatter16 = _make_scatter_kernel(16)
_scatter32 = _make_scatter_kernel(32)


def _tc0_body(degp_ref, dinv_ref):
    d = degp_ref[...]
    deg = jnp.sum(d, axis=0) + 1.0
    dinv_ref[...] = lax.rsqrt(deg)


def _tc0(degp):
    return pl.pallas_call(
        _tc0_body,
        out_shape=jax.ShapeDtypeStruct((ROWS,), jnp.float32),
    )(degp)


def _tc1_body(dinv_ref, x_ref, w_ref, g_ref):
    dinv16 = jnp.broadcast_to(dinv_ref[...], (N, 16))
    h = jnp.dot(x_ref[...], w_ref[...], preferred_element_type=jnp.float32)
    g_ref[0:N, :] = dinv16 * h


def _tc_mid_body(s_ref, g_ref, dinv_ref, w_ref, b_ref, o_ref):
    sp = s_ref[...]
    ssum = sp[0, :N, :] + sp[1, :N, :]
    g = g_ref[...][:N, :]
    dinv = jnp.broadcast_to(dinv_ref[...], (N, 16))
    u = jnp.maximum(dinv * (ssum + g) + b_ref[...], 0.0)
    h = jnp.dot(u, w_ref[...], preferred_element_type=jnp.float32)
    fo = h.shape[1]
    dinv_o = jnp.broadcast_to(dinv_ref[...], (N, fo))
    o_ref[0:N, :] = dinv_o * h


def _tc_out_body(s_ref, g_ref, dinv_ref, b_ref, o_ref):
    sp = s_ref[...]
    ssum = sp[0, :N, :] + sp[1, :N, :]
    g = g_ref[...][:N, :]
    dinv = jnp.broadcast_to(dinv_ref[...], (N, 32))
    o_ref[...] = jax.nn.sigmoid(dinv * (ssum + g) + b_ref[...])


def _tc1(dinv_col, x, w1):
    return pl.pallas_call(
        _tc1_body,
        out_shape=jax.ShapeDtypeStruct((ROWS, 16), jnp.float32),
    )(dinv_col, x, w1)


def _tc_mid(s, g, dinv, w, b, fo):
    return pl.pallas_call(
        _tc_mid_body,
        out_shape=jax.ShapeDtypeStruct((ROWS, fo), jnp.float32),
    )(s, g, dinv, w, b)


def _tc_out(s, g, dinv, b):
    return pl.pallas_call(
        _tc_out_body,
        out_shape=jax.ShapeDtypeStruct((N, 32), jnp.float32),
    )(s, g, dinv, b)


def kernel(x, edge_index, W1, b1, W2, b2, W3, b3):
    src = edge_index[0].astype(jnp.int32)
    dst = edge_index[1].astype(jnp.int32)
    npad = EPAD - E
    srcp = jnp.concatenate([src, jnp.zeros((npad,), jnp.int32)]).reshape(NW, NBLK, BLK)
    dstp = jnp.concatenate([dst, jnp.full((npad,), ROWS - 1, jnp.int32)]).reshape(NW, NBLK, BLK)

    degp = _deg_kernel(dstp)
    dinv_flat = _tc0(degp)
    dinv_col = dinv_flat.reshape(ROWS, 1)[:N]
    g1 = _tc1(dinv_col, x, W1)
    s1 = _scatter16(g1, srcp, dstp)
    g2 = _tc_mid(s1, g1, dinv_col, W2, b1.reshape(1, 16), 16)
    s2 = _scatter16(g2, srcp, dstp)
    g3 = _tc_mid(s2, g2, dinv_col, W3, b2.reshape(1, 16), 32)
    s3 = _scatter32(g3, srcp, dstp)
    return _tc_out(s3, g3, dinv_col, b3.reshape(1, 32))


# speedup vs baseline: 40.6282x; 1.1355x over previous
import functools

import jax
import jax.numpy as jnp
from jax import lax
from jax.experimental import pallas as pl
from jax.experimental.pallas import tpu as pltpu
from jax.experimental.pallas import tpu_sc as plsc

N = 10000
E = 640000
NC = 2
NS = 16
NW = NC * NS
BLK = 128
NBLK = 157
EPAD = NW * NBLK * BLK
ROWS = 10112
SLICE = ROWS // NS

_MESH = plsc.VectorSubcoreMesh(core_axis_name="c", subcore_axis_name="s")


def _zero_rows(zbuf, nrows, f):
    def body(i, _):
        for f0 in range(0, f, 16):
            zbuf[i, f0:f0 + 16] = jnp.zeros((16,), jnp.float32)
        return 0
    lax.fori_loop(0, nrows, body, 0)


HR = ROWS // 128


def _make_deg_kernel():
    @functools.partial(
        pl.kernel,
        out_type=jax.ShapeDtypeStruct((NW, ROWS), jnp.float32),
        mesh=_MESH,
        scratch_types=[
            pltpu.VMEM((NBLK, BLK), jnp.int32),
            pltpu.VMEM((ROWS,), jnp.float32),
        ],
        compiler_params=pltpu.CompilerParams(needs_layout_passes=False),
    )
    def deg_kernel(dstp_hbm, out_hbm, dst_v, hist):
        c = lax.axis_index("c")
        s = lax.axis_index("s")
        wid = c * NS + s
        def z(i, _):
            hist[pl.ds(i * 16, 16)] = jnp.zeros((16,), jnp.float32)
            return 0
        lax.fori_loop(0, ROWS // 16, z, 0)
        pltpu.sync_copy(dstp_hbm.at[wid], dst_v)
        ones = jnp.ones((16,), jnp.float32)
        def body(j, _):
            for k in range(0, BLK, 16):
                d = dst_v[j, k:k + 16]
                plsc.addupdate_scatter(hist, [d], ones)
            return 0
        lax.fori_loop(0, NBLK, body, 0)
        pltpu.sync_copy(hist, out_hbm.at[wid])

    return deg_kernel


def _make_scatter_kernel(f):
    @functools.partial(
        pl.kernel,
        out_type=jax.ShapeDtypeStruct((NC, ROWS, f), jnp.float32),
        mesh=_MESH,
        scratch_types=[
            pltpu.VMEM((NBLK, BLK), jnp.int32),
            pltpu.VMEM((NBLK, BLK), jnp.int32),
            pltpu.VMEM((BLK, f), jnp.float32),
            pltpu.VMEM((BLK, f), jnp.float32),
            pltpu.VMEM((SLICE, f), jnp.float32),
            pltpu.VMEM_SHARED((ROWS, f), jnp.float32),
            pltpu.SemaphoreType.DMA,
            pltpu.SemaphoreType.DMA,
        ],
        compiler_params=pltpu.CompilerParams(use_tc_tiling_on_sc=False),
    )
    def scatter_kernel(g_hbm, srcp_hbm, dstp_hbm, out_hbm,
                       src_v, dst_v, rows_a, rows_b, zbuf, acc, sem_a, sem_b):
        c = lax.axis_index("c")
        s = lax.axis_index("s")
        wid = c * NS + s
        _zero_rows(zbuf, SLICE, f)
        pltpu.sync_copy(zbuf, acc.at[pl.ds(s * SLICE, SLICE)])
        pltpu.sync_copy(srcp_hbm.at[wid], src_v)
        pltpu.sync_copy(dstp_hbm.at[wid], dst_v)
        plsc.subcore_barrier()
        pltpu.async_copy(g_hbm.at[src_v.at[0]], rows_a, sem_a)
        def pair(t, _):
            j = 2 * t
            pltpu.make_async_copy(g_hbm.at[src_v.at[j]], rows_a, sem_a).wait()
            pltpu.async_copy(g_hbm.at[src_v.at[j + 1]], rows_b, sem_b)
            pltpu.sync_copy(rows_a, acc.at[dst_v.at[j]], add=True)
            pltpu.make_async_copy(g_hbm.at[src_v.at[j + 1]], rows_b, sem_b).wait()
            pltpu.async_copy(g_hbm.at[src_v.at[j + 2]], rows_a, sem_a)
            pltpu.sync_copy(rows_b, acc.at[dst_v.at[j + 1]], add=True)
            return 0
        lax.fori_loop(0, (NBLK - 1) // 2, pair, 0)
        pltpu.make_async_copy(g_hbm.at[src_v.at[NBLK - 1]], rows_a, sem_a).wait()
        pltpu.sync_copy(rows_a, acc.at[dst_v.at[NBLK - 1]], add=True)
        plsc.subcore_barrier()
        pltpu.sync_copy(acc.at[pl.ds(s * SLICE, SLICE)],
                        out_hbm.at[c, pl.ds(s * SLICE, SLICE)])

    return scatter_kernel


_deg_kernel = _make_deg_kernel()
_scatter16 = _make_scatter_kernel(16)
_scatter32 = _make_scatter_kernel(32)


def _tc0_body(degp_ref, dinv_ref):
    d = degp_ref[...]
    deg = jnp.sum(d, axis=0) + 1.0
    dinv_ref[...] = lax.rsqrt(deg)


def _tc0(degp):
    return pl.pallas_call(
        _tc0_body,
        out_shape=jax.ShapeDtypeStruct((ROWS,), jnp.float32),
    )(degp)


def _tc1_body(dinv_ref, x_ref, w_ref, g_ref):
    dinv16 = jnp.broadcast_to(dinv_ref[...], (N, 16))
    h = jnp.dot(x_ref[...], w_ref[...], preferred_element_type=jnp.float32)
    g_ref[0:N, :] = dinv16 * h


def _tc_mid_body(s_ref, g_ref, dinv_ref, w_ref, b_ref, o_ref):
    sp = s_ref[...]
    ssum = sp[0, :N, :] + sp[1, :N, :]
    g = g_ref[...][:N, :]
    dinv = jnp.broadcast_to(dinv_ref[...], (N, 16))
    u = jnp.maximum(dinv * (ssum + g) + b_ref[...], 0.0)
    h = jnp.dot(u, w_ref[...], preferred_element_type=jnp.float32)
    fo = h.shape[1]
    dinv_o = jnp.broadcast_to(dinv_ref[...], (N, fo))
    o_ref[0:N, :] = dinv_o * h


def _tc_out_body(s_ref, g_ref, dinv_ref, b_ref, o_ref):
    sp = s_ref[...]
    ssum = sp[0, :N, :] + sp[1, :N, :]
    g = g_ref[...][:N, :]
    dinv = jnp.broadcast_to(dinv_ref[...], (N, 32))
    o_ref[...] = jax.nn.sigmoid(dinv * (ssum + g) + b_ref[...])


def _tc1(dinv_col, x, w1):
    return pl.pallas_call(
        _tc1_body,
        out_shape=jax.ShapeDtypeStruct((ROWS, 16), jnp.float32),
    )(dinv_col, x, w1)


def _tc_mid(s, g, dinv, w, b, fo):
    return pl.pallas_call(
        _tc_mid_body,
        out_shape=jax.ShapeDtypeStruct((ROWS, fo), jnp.float32),
    )(s, g, dinv, w, b)


def _tc_out(s, g, dinv, b):
    return pl.pallas_call(
        _tc_out_body,
        out_shape=jax.ShapeDtypeStruct((N, 32), jnp.float32),
    )(s, g, dinv, b)


def kernel(x, edge_index, W1, b1, W2, b2, W3, b3):
    src = edge_index[0].astype(jnp.int32)
    dst = edge_index[1].astype(jnp.int32)
    npad = EPAD - E
    srcp = jnp.concatenate([src, jnp.zeros((npad,), jnp.int32)]).reshape(NW, NBLK, BLK)
    dstp = jnp.concatenate([dst, jnp.full((npad,), ROWS - 1, jnp.int32)]).reshape(NW, NBLK, BLK)

    degp = _deg_kernel(dstp)
    dinv_flat = _tc0(degp)
    dinv_col = dinv_flat.reshape(ROWS, 1)[:N]
    g1 = _tc1(dinv_col, x, W1)
    s1 = _scatter16(g1, srcp, dstp)
    g2 = _tc_mid(s1, g1, dinv_col, W2, b1.reshape(1, 16), 16)
    s2 = _scatter16(g2, srcp, dstp)
    g3 = _tc_mid(s2, g2, dinv_col, W3, b2.reshape(1, 16), 32)
    s3 = _scatter32(g3, srcp, dstp)
    return _tc_out(s3, g3, dinv_col, b3.reshape(1, 32))


# speedup vs baseline: 69.3216x; 1.7062x over previous
import functools

import jax
import jax.numpy as jnp
from jax import lax
from jax.experimental import pallas as pl
from jax.experimental.pallas import tpu as pltpu
from jax.experimental.pallas import tpu_sc as plsc

N = 10000
E = 640000
NC = 2
NS = 16
NW = NC * NS
BLK = 128
NBLK = 157
EPAD = NW * NBLK * BLK
ROWS = 10112
SLICE = ROWS // NS

_MESH = plsc.VectorSubcoreMesh(core_axis_name="c", subcore_axis_name="s")


def _zero_rows(zbuf, nrows, f):
    def body(i, _):
        for f0 in range(0, f, 16):
            zbuf[i, f0:f0 + 16] = jnp.zeros((16,), jnp.float32)
        return 0
    lax.fori_loop(0, nrows, body, 0)


HR = ROWS // 128


def _make_deg_kernel():
    @functools.partial(
        pl.kernel,
        out_type=jax.ShapeDtypeStruct((NW, ROWS), jnp.float32),
        mesh=_MESH,
        scratch_types=[
            pltpu.VMEM((NBLK, BLK), jnp.int32),
            pltpu.VMEM((ROWS,), jnp.float32),
        ],
        compiler_params=pltpu.CompilerParams(needs_layout_passes=False),
    )
    def deg_kernel(dstp_hbm, out_hbm, dst_v, hist):
        c = lax.axis_index("c")
        s = lax.axis_index("s")
        wid = c * NS + s
        def z(i, _):
            hist[pl.ds(i * 16, 16)] = jnp.zeros((16,), jnp.float32)
            return 0
        lax.fori_loop(0, ROWS // 16, z, 0)
        pltpu.sync_copy(dstp_hbm.at[wid], dst_v)
        ones = jnp.ones((16,), jnp.float32)
        def body(j, _):
            for k in range(0, BLK, 16):
                d = dst_v[j, k:k + 16]
                plsc.addupdate_scatter(hist, [d], ones)
            return 0
        lax.fori_loop(0, NBLK, body, 0)
        pltpu.sync_copy(hist, out_hbm.at[wid])

    return deg_kernel


def _make_scatter_kernel(f):
    @functools.partial(
        pl.kernel,
        out_type=jax.ShapeDtypeStruct((NC, ROWS, f), jnp.float32),
        mesh=_MESH,
        scratch_types=[
            pltpu.VMEM((NBLK, BLK), jnp.int32),
            pltpu.VMEM((NBLK, BLK), jnp.int32),
            pltpu.VMEM((BLK, f), jnp.float32),
            pltpu.VMEM((BLK, f), jnp.float32),
            pltpu.VMEM((SLICE, f), jnp.float32),
            pltpu.VMEM_SHARED((ROWS, f), jnp.float32),
            pltpu.VMEM_SHARED((ROWS, f), jnp.float32),
            pltpu.SemaphoreType.DMA,
            pltpu.SemaphoreType.DMA,
        ],
        compiler_params=pltpu.CompilerParams(use_tc_tiling_on_sc=False),
    )
    def scatter_kernel(g_hbm, srcp_hbm, dstp_hbm, out_hbm,
                       src_v, dst_v, rows_a, rows_b, zbuf, acc, gtab,
                       sem_a, sem_b):
        c = lax.axis_index("c")
        s = lax.axis_index("s")
        wid = c * NS + s
        _zero_rows(zbuf, SLICE, f)
        pltpu.sync_copy(zbuf, acc.at[pl.ds(s * SLICE, SLICE)])
        pltpu.sync_copy(g_hbm.at[pl.ds(s * SLICE, SLICE)],
                        gtab.at[pl.ds(s * SLICE, SLICE)])
        pltpu.sync_copy(srcp_hbm.at[wid], src_v)
        pltpu.sync_copy(dstp_hbm.at[wid], dst_v)
        plsc.subcore_barrier()
        pltpu.async_copy(gtab.at[src_v.at[0]], rows_a, sem_a)
        def pair(t, _):
            j = 2 * t
            pltpu.make_async_copy(gtab.at[src_v.at[j]], rows_a, sem_a).wait()
            pltpu.async_copy(gtab.at[src_v.at[j + 1]], rows_b, sem_b)
            pltpu.sync_copy(rows_a, acc.at[dst_v.at[j]], add=True)
            pltpu.make_async_copy(gtab.at[src_v.at[j + 1]], rows_b, sem_b).wait()
            pltpu.async_copy(gtab.at[src_v.at[j + 2]], rows_a, sem_a)
            pltpu.sync_copy(rows_b, acc.at[dst_v.at[j + 1]], add=True)
            return 0
        lax.fori_loop(0, (NBLK - 1) // 2, pair, 0)
        pltpu.make_async_copy(gtab.at[src_v.at[NBLK - 1]], rows_a, sem_a).wait()
        pltpu.sync_copy(rows_a, acc.at[dst_v.at[NBLK - 1]], add=True)
        plsc.subcore_barrier()
        pltpu.sync_copy(acc.at[pl.ds(s * SLICE, SLICE)],
                        out_hbm.at[c, pl.ds(s * SLICE, SLICE)])

    return scatter_kernel


_deg_kernel = _make_deg_kernel()
_scatter16 = _make_scatter_kernel(16)


def _tc0_body(degp_ref, dinv_ref):
    d = degp_ref[...]
    deg = jnp.sum(d, axis=0) + 1.0
    dinv_ref[...] = lax.rsqrt(deg)


def _tc0(degp):
    return pl.pallas_call(
        _tc0_body,
        out_shape=jax.ShapeDtypeStruct((ROWS,), jnp.float32),
    )(degp)


def _tc1_body(dinv_ref, x_ref, w_ref, g_ref):
    dinv16 = jnp.broadcast_to(dinv_ref[...], (N, 16))
    h = jnp.dot(x_ref[...], w_ref[...], preferred_element_type=jnp.float32)
    g_ref[0:N, :] = dinv16 * h


def _tc_mid_body(s_ref, g_ref, dinv_ref, w_ref, b_ref, o_ref):
    sp = s_ref[...]
    ssum = sp[0, :N, :] + sp[1, :N, :]
    g = g_ref[...][:N, :]
    dinv = jnp.broadcast_to(dinv_ref[...], (N, 16))
    u = jnp.maximum(dinv * (ssum + g) + b_ref[...], 0.0)
    h = jnp.dot(u, w_ref[...], preferred_element_type=jnp.float32)
    fo = h.shape[1]
    dinv_o = jnp.broadcast_to(dinv_ref[...], (N, fo))
    o_ref[0:N, :] = dinv_o * h


def _tc_mid3_body(s_ref, g_ref, dinv_ref, w_ref, b_ref, oa_ref, ob_ref):
    sp = s_ref[...]
    ssum = sp[0, :N, :] + sp[1, :N, :]
    g = g_ref[...][:N, :]
    dinv = jnp.broadcast_to(dinv_ref[...], (N, 16))
    u = jnp.maximum(dinv * (ssum + g) + b_ref[...], 0.0)
    h = jnp.dot(u, w_ref[...], preferred_element_type=jnp.float32)
    dinv_o = jnp.broadcast_to(dinv_ref[...], (N, 32))
    g3 = dinv_o * h
    oa_ref[0:N, :] = g3[:, 0:16]
    ob_ref[0:N, :] = g3[:, 16:32]


def _tc_out_body(sa_ref, sb_ref, ga_ref, gb_ref, dinv_ref, b_ref, o_ref):
    sa = sa_ref[...]
    sb = sb_ref[...]
    ssum = jnp.concatenate(
        [sa[0, :N, :] + sa[1, :N, :], sb[0, :N, :] + sb[1, :N, :]], axis=1)
    g = jnp.concatenate([ga_ref[...][:N, :], gb_ref[...][:N, :]], axis=1)
    dinv = jnp.broadcast_to(dinv_ref[...], (N, 32))
    o_ref[...] = jax.nn.sigmoid(dinv * (ssum + g) + b_ref[...])


def _tc1(dinv_col, x, w1):
    return pl.pallas_call(
        _tc1_body,
        out_shape=jax.ShapeDtypeStruct((ROWS, 16), jnp.float32),
    )(dinv_col, x, w1)


def _tc_mid(s, g, dinv, w, b):
    return pl.pallas_call(
        _tc_mid_body,
        out_shape=jax.ShapeDtypeStruct((ROWS, 16), jnp.float32),
    )(s, g, dinv, w, b)


def _tc_mid3(s, g, dinv, w, b):
    return pl.pallas_call(
        _tc_mid3_body,
        out_shape=[jax.ShapeDtypeStruct((ROWS, 16), jnp.float32),
                   jax.ShapeDtypeStruct((ROWS, 16), jnp.float32)],
    )(s, g, dinv, w, b)


def _tc_out(sa, sb, ga, gb, dinv, b):
    return pl.pallas_call(
        _tc_out_body,
        out_shape=jax.ShapeDtypeStruct((N, 32), jnp.float32),
    )(sa, sb, ga, gb, dinv, b)


def kernel(x, edge_index, W1, b1, W2, b2, W3, b3):
    src = edge_index[0].astype(jnp.int32)
    dst = edge_index[1].astype(jnp.int32)
    npad = EPAD - E
    srcp = jnp.concatenate([src, jnp.zeros((npad,), jnp.int32)]).reshape(NW, NBLK, BLK)
    dstp = jnp.concatenate([dst, jnp.full((npad,), ROWS - 1, jnp.int32)]).reshape(NW, NBLK, BLK)

    degp = _deg_kernel(dstp)
    dinv_flat = _tc0(degp)
    dinv_col = dinv_flat.reshape(ROWS, 1)[:N]
    g1 = _tc1(dinv_col, x, W1)
    s1 = _scatter16(g1, srcp, dstp)
    g2 = _tc_mid(s1, g1, dinv_col, W2, b1.reshape(1, 16))
    s2 = _scatter16(g2, srcp, dstp)
    g3a, g3b = _tc_mid3(s2, g2, dinv_col, W3, b2.reshape(1, 16))
    s3a = _scatter16(g3a, srcp, dstp)
    s3b = _scatter16(g3b, srcp, dstp)
    return _tc_out(s3a, s3b, g3a, g3b, dinv_col, b3.reshape(1, 32))
